# baseline (speedup 1.0000x reference)
import functools

import jax
import jax.numpy as jnp
from jax import lax
from jax.experimental import pallas as pl
from jax.experimental.pallas import tpu as pltpu
from jax.experimental.pallas import tpu_sc as plsc

D_MODEL = 1024
BATCH = 16384
S_TC = 12288
BS_TC = 512


@functools.partial(jax.jit, static_argnames=())
def _sc_embed(table, idx):
    info = plsc.get_sparse_core_info()
    nc, ns = info.num_cores, info.num_subcores
    nw = nc * ns
    b_per_w = (BATCH - S_TC) // nw
    ch = 48
    nbuf = 2
    n_chunks = -(-b_per_w // ch)
    sizes = [min(ch, b_per_w - g * ch) for g in range(n_chunks)]

    mesh = plsc.VectorSubcoreMesh(core_axis_name="c", subcore_axis_name="s")

    @functools.partial(
        pl.kernel,
        mesh=mesh,
        out_type=(
            jax.ShapeDtypeStruct((BATCH, 1, D_MODEL), jnp.float32),
            jax.ShapeDtypeStruct((16 * nw, 1, D_MODEL), jnp.float32),
        ),
        scratch_types=[
            pltpu.VMEM((b_per_w,), jnp.int32),
            pltpu.VMEM((13, 1, D_MODEL), jnp.float32),
            pltpu.VMEM((nbuf, ch, 1, D_MODEL), jnp.float32),
            pltpu.SemaphoreType.DMA,
            pltpu.SemaphoreType.DMA,
        ],
    )
    def k(table_hbm, idx_hbm, out_hbm, rep_hbm, idx_v, tab_v, rows_v, gsem,
          ssem):
        wid = lax.axis_index("s") * nc + lax.axis_index("c")
        base = S_TC + wid * b_per_w
        pltpu.sync_copy(table_hbm, tab_v)
        pltpu.sync_copy(tab_v, rep_hbm.at[pl.ds(wid * 16, 13)])
        pltpu.sync_copy(idx_hbm.at[pl.ds(base, b_per_w)], idx_v)
        off = jnp.broadcast_to((wid * 16).astype(jnp.int32), (16,))
        for j in range(b_per_w // 16):
            sl = pl.ds(j * 16, 16)
            idx_v[sl] = idx_v[sl] + off

        def start_gather(g):
            return pltpu.async_copy(
                rep_hbm.at[idx_v.at[pl.ds(g * ch, sizes[g])]],
                rows_v.at[g % nbuf].at[pl.ds(0, sizes[g])], gsem)

        ga = [None] * n_chunks
        sc = [None] * n_chunks
        for g in range(min(nbuf - 1, n_chunks)):
            ga[g] = start_gather(g)
        for g in range(n_chunks):
            nxt = g + nbuf - 1
            if nxt < n_chunks:
                if g >= 1:
                    sc[g - 1].wait()
                ga[nxt] = start_gather(nxt)
            ga[g].wait()
            sc[g] = pltpu.async_copy(
                rows_v.at[g % nbuf].at[pl.ds(0, sizes[g])],
                out_hbm.at[pl.ds(base + g * ch, sizes[g])], ssem)
        for g in range(max(0, n_chunks - nbuf), n_chunks):
            sc[g].wait()

    return k(table, idx)[0]


def _tc_fill(wk3, table, buf):
    def body(wk_ref, tab_ref, alias_ref, out_ref):
        del alias_ref
        wk = wk_ref[0, 0, :]
        oh = (wk[:, None] == lax.broadcasted_iota(jnp.int32, (BS_TC, 13), 1))
        out_ref[:, 0, :] = jax.lax.dot_general(
            oh.astype(jnp.float32), tab_ref[...],
            dimension_numbers=(((1,), (0,)), ((), ())),
            preferred_element_type=jnp.float32)

    return pl.pallas_call(
        body,
        grid=(S_TC // BS_TC,),
        in_specs=[
            pl.BlockSpec((1, 1, BS_TC), lambda i: (i, 0, 0)),
            pl.BlockSpec((13, D_MODEL), lambda i: (0, 0)),
            pl.BlockSpec(memory_space=pl.ANY),
        ],
        out_specs=pl.BlockSpec((BS_TC, 1, D_MODEL), lambda i: (i, 0, 0)),
        out_shape=jax.ShapeDtypeStruct((BATCH, 1, D_MODEL), jnp.float32),
        input_output_aliases={2: 0},
    )(wk3, table, buf)


def kernel(weeks, table):
    wk = weeks.astype(jnp.int32)
    buf = _sc_embed(table[:, None, :], wk)
    wk3 = wk[:S_TC].reshape(S_TC // BS_TC, 1, BS_TC)
    return _tc_fill(wk3, table, buf)


# speedup vs baseline: 1.0429x; 1.0429x over previous
import functools

import jax
import jax.numpy as jnp
from jax import lax
from jax.experimental import pallas as pl
from jax.experimental.pallas import tpu as pltpu
from jax.experimental.pallas import tpu_sc as plsc

D_MODEL = 1024
BATCH = 16384
S_TC = 12288
BS_TC = 512


@functools.partial(jax.jit, static_argnames=())
def _sc_embed(table, idx):
    info = plsc.get_sparse_core_info()
    nc, ns = info.num_cores, info.num_subcores
    nw = nc * ns
    b_per_w = (BATCH - S_TC) // nw
    ch = 48
    nbuf = 2
    n_chunks = -(-b_per_w // ch)
    sizes = [min(ch, b_per_w - g * ch) for g in range(n_chunks)]

    mesh = plsc.VectorSubcoreMesh(core_axis_name="c", subcore_axis_name="s")

    @functools.partial(
        pl.kernel,
        mesh=mesh,
        out_type=jax.ShapeDtypeStruct((BATCH, 1, D_MODEL), jnp.float32),
        scratch_types=[
            pltpu.VMEM((b_per_w,), jnp.int32),
            pltpu.VMEM((nbuf, ch, 1, D_MODEL), jnp.float32),
            pltpu.SemaphoreType.DMA,
            pltpu.SemaphoreType.DMA,
        ],
    )
    def k(table_hbm, idx_hbm, out_hbm, idx_v, rows_v, gsem, ssem):
        wid = lax.axis_index("s") * nc + lax.axis_index("c")
        base = S_TC + wid * b_per_w
        pltpu.sync_copy(idx_hbm.at[pl.ds(base, b_per_w)], idx_v)
        off = jnp.broadcast_to((wid * 16).astype(jnp.int32), (16,))
        for j in range(b_per_w // 16):
            sl = pl.ds(j * 16, 16)
            idx_v[sl] = idx_v[sl] + off

        def start_gather(g):
            return pltpu.async_copy(
                table_hbm.at[idx_v.at[pl.ds(g * ch, sizes[g])]],
                rows_v.at[g % nbuf].at[pl.ds(0, sizes[g])], gsem)

        ga = [None] * n_chunks
        sc = [None] * n_chunks
        for g in range(min(nbuf - 1, n_chunks)):
            ga[g] = start_gather(g)
        for g in range(n_chunks):
            nxt = g + nbuf - 1
            if nxt < n_chunks:
                if g >= 1:
                    sc[g - 1].wait()
                ga[nxt] = start_gather(nxt)
            ga[g].wait()
            sc[g] = pltpu.async_copy(
                rows_v.at[g % nbuf].at[pl.ds(0, sizes[g])],
                out_hbm.at[pl.ds(base + g * ch, sizes[g])], ssem)
        for g in range(max(0, n_chunks - nbuf), n_chunks):
            sc[g].wait()

    return k(table, idx)


def _tc_replicate(table):
    def body(tab_ref, out_ref):
        for r in range(32):
            out_ref[pl.ds(r * 16, 13), 0, :] = tab_ref[...]

    return pl.pallas_call(
        body,
        out_shape=jax.ShapeDtypeStruct((512, 1, D_MODEL), jnp.float32),
    )(table)


def _tc_fill(wk3, table, buf):
    def body(wk_ref, tab_ref, alias_ref, out_ref):
        del alias_ref
        wk = wk_ref[0, 0, :]
        oh = (wk[:, None] == lax.broadcasted_iota(jnp.int32, (BS_TC, 13), 1))
        out_ref[:, 0, :] = jax.lax.dot_general(
            oh.astype(jnp.float32), tab_ref[...],
            dimension_numbers=(((1,), (0,)), ((), ())),
            preferred_element_type=jnp.float32)

    return pl.pallas_call(
        body,
        grid=(S_TC // BS_TC,),
        in_specs=[
            pl.BlockSpec((1, 1, BS_TC), lambda i: (i, 0, 0)),
            pl.BlockSpec((13, D_MODEL), lambda i: (0, 0)),
            pl.BlockSpec(memory_space=pl.ANY),
        ],
        out_specs=pl.BlockSpec((BS_TC, 1, D_MODEL), lambda i: (i, 0, 0)),
        out_shape=jax.ShapeDtypeStruct((BATCH, 1, D_MODEL), jnp.float32),
        input_output_aliases={2: 0},
    )(wk3, table, buf)


def kernel(weeks, table):
    wk = weeks.astype(jnp.int32)
    rep = _tc_replicate(table)
    buf = _sc_embed(rep, wk)
    wk3 = wk[:S_TC].reshape(S_TC // BS_TC, 1, BS_TC)
    return _tc_fill(wk3, table, buf)


# speedup vs baseline: 1.1396x; 1.0927x over previous
import functools

import jax
import jax.numpy as jnp
from jax import lax
from jax.experimental import pallas as pl
from jax.experimental.pallas import tpu as pltpu
from jax.experimental.pallas import tpu_sc as plsc

D_MODEL = 1024
BATCH = 16384
S_TC = 12288
BS_TC = 1024


@functools.partial(jax.jit, static_argnames=())
def _sc_embed(table, idx):
    info = plsc.get_sparse_core_info()
    nc, ns = info.num_cores, info.num_subcores
    nw = nc * ns
    b_per_w = (BATCH - S_TC) // nw
    ch = 48
    nbuf = 2
    n_chunks = -(-b_per_w // ch)
    sizes = [min(ch, b_per_w - g * ch) for g in range(n_chunks)]

    mesh = plsc.VectorSubcoreMesh(core_axis_name="c", subcore_axis_name="s")

    @functools.partial(
        pl.kernel,
        mesh=mesh,
        out_type=jax.ShapeDtypeStruct((BATCH, 1, D_MODEL), jnp.float32),
        scratch_types=[
            pltpu.VMEM((b_per_w,), jnp.int32),
            pltpu.VMEM((nbuf, ch, 1, D_MODEL), jnp.float32),
            pltpu.SemaphoreType.DMA,
            pltpu.SemaphoreType.DMA,
        ],
    )
    def k(table_hbm, idx_hbm, out_hbm, idx_v, rows_v, gsem, ssem):
        wid = lax.axis_index("s") * nc + lax.axis_index("c")
        base = S_TC + wid * b_per_w
        pltpu.sync_copy(idx_hbm.at[pl.ds(base, b_per_w)], idx_v)
        off = jnp.broadcast_to((wid * 16).astype(jnp.int32), (16,))
        for j in range(b_per_w // 16):
            sl = pl.ds(j * 16, 16)
            idx_v[sl] = idx_v[sl] + off

        def start_gather(g):
            return pltpu.async_copy(
                table_hbm.at[idx_v.at[pl.ds(g * ch, sizes[g])]],
                rows_v.at[g % nbuf].at[pl.ds(0, sizes[g])], gsem)

        ga = [None] * n_chunks
        sc = [None] * n_chunks
        for g in range(min(nbuf - 1, n_chunks)):
            ga[g] = start_gather(g)
        for g in range(n_chunks):
            nxt = g + nbuf - 1
            if nxt < n_chunks:
                if g >= 1:
                    sc[g - 1].wait()
                ga[nxt] = start_gather(nxt)
            ga[g].wait()
            sc[g] = pltpu.async_copy(
                rows_v.at[g % nbuf].at[pl.ds(0, sizes[g])],
                out_hbm.at[pl.ds(base + g * ch, sizes[g])], ssem)
        for g in range(max(0, n_chunks - nbuf), n_chunks):
            sc[g].wait()

    return k(table, idx)


def _tc_replicate(table):
    def body(tab_ref, out_ref):
        for r in range(32):
            out_ref[pl.ds(r * 16, 13), 0, :] = tab_ref[...]

    return pl.pallas_call(
        body,
        out_shape=jax.ShapeDtypeStruct((512, 1, D_MODEL), jnp.float32),
    )(table)


def _tc_fill(wk3, table, buf):
    def body(wk_ref, tab_ref, alias_ref, out_ref):
        del alias_ref
        wk = wk_ref[0, 0, :]
        oh = (wk[:, None] == lax.broadcasted_iota(jnp.int32, (BS_TC, 13), 1))
        out_ref[:, 0, :] = jax.lax.dot_general(
            oh.astype(jnp.float32), tab_ref[...],
            dimension_numbers=(((1,), (0,)), ((), ())),
            preferred_element_type=jnp.float32)

    return pl.pallas_call(
        body,
        grid=(S_TC // BS_TC,),
        in_specs=[
            pl.BlockSpec((1, 1, BS_TC), lambda i: (i, 0, 0)),
            pl.BlockSpec((13, D_MODEL), lambda i: (0, 0)),
            pl.BlockSpec(memory_space=pl.ANY),
        ],
        out_specs=pl.BlockSpec((BS_TC, 1, D_MODEL), lambda i: (i, 0, 0)),
        out_shape=jax.ShapeDtypeStruct((BATCH, 1, D_MODEL), jnp.float32),
        input_output_aliases={2: 0},
    )(wk3, table, buf)


def kernel(weeks, table):
    wk = weeks.astype(jnp.int32)
    rep = _tc_replicate(table)
    buf = _sc_embed(rep, wk)
    wk3 = wk[:S_TC].reshape(S_TC // BS_TC, 1, BS_TC)
    return _tc_fill(wk3, table, buf)


# speedup vs baseline: 1.1626x; 1.0202x over previous
import functools

import jax
import jax.numpy as jnp
from jax import lax
from jax.experimental import pallas as pl
from jax.experimental.pallas import tpu as pltpu
from jax.experimental.pallas import tpu_sc as plsc

D_MODEL = 1024
BATCH = 16384
S_TC = 12288
BS_TC = 2048


@functools.partial(jax.jit, static_argnames=())
def _sc_embed(table, idx):
    info = plsc.get_sparse_core_info()
    nc, ns = info.num_cores, info.num_subcores
    nw = nc * ns
    b_per_w = (BATCH - S_TC) // nw
    ch = 48
    nbuf = 2
    n_chunks = -(-b_per_w // ch)
    sizes = [min(ch, b_per_w - g * ch) for g in range(n_chunks)]

    mesh = plsc.VectorSubcoreMesh(core_axis_name="c", subcore_axis_name="s")

    @functools.partial(
        pl.kernel,
        mesh=mesh,
        out_type=jax.ShapeDtypeStruct((BATCH, 1, D_MODEL), jnp.float32),
        scratch_types=[
            pltpu.VMEM((b_per_w,), jnp.int32),
            pltpu.VMEM((nbuf, ch, 1, D_MODEL), jnp.float32),
            pltpu.SemaphoreType.DMA,
            pltpu.SemaphoreType.DMA,
        ],
    )
    def k(table_hbm, idx_hbm, out_hbm, idx_v, rows_v, gsem, ssem):
        wid = lax.axis_index("s") * nc + lax.axis_index("c")
        base = S_TC + wid * b_per_w
        pltpu.sync_copy(idx_hbm.at[pl.ds(base, b_per_w)], idx_v)
        off = jnp.broadcast_to((wid * 16).astype(jnp.int32), (16,))
        for j in range(b_per_w // 16):
            sl = pl.ds(j * 16, 16)
            idx_v[sl] = idx_v[sl] + off

        def start_gather(g):
            return pltpu.async_copy(
                table_hbm.at[idx_v.at[pl.ds(g * ch, sizes[g])]],
                rows_v.at[g % nbuf].at[pl.ds(0, sizes[g])], gsem)

        ga = [None] * n_chunks
        sc = [None] * n_chunks
        for g in range(min(nbuf - 1, n_chunks)):
            ga[g] = start_gather(g)
        for g in range(n_chunks):
            nxt = g + nbuf - 1
            if nxt < n_chunks:
                if g >= 1:
                    sc[g - 1].wait()
                ga[nxt] = start_gather(nxt)
            ga[g].wait()
            sc[g] = pltpu.async_copy(
                rows_v.at[g % nbuf].at[pl.ds(0, sizes[g])],
                out_hbm.at[pl.ds(base + g * ch, sizes[g])], ssem)
        for g in range(max(0, n_chunks - nbuf), n_chunks):
            sc[g].wait()

    return k(table, idx)


def _tc_replicate(table):
    def body(tab_ref, out_ref):
        for r in range(32):
            out_ref[pl.ds(r * 16, 13), 0, :] = tab_ref[...]

    return pl.pallas_call(
        body,
        out_shape=jax.ShapeDtypeStruct((512, 1, D_MODEL), jnp.float32),
    )(table)


def _tc_fill(wk3, table, buf):
    def body(wk_ref, tab_ref, alias_ref, out_ref):
        del alias_ref
        wk = wk_ref[0, 0, :]
        oh = (wk[:, None] == lax.broadcasted_iota(jnp.int32, (BS_TC, 13), 1))
        out_ref[:, 0, :] = jax.lax.dot_general(
            oh.astype(jnp.float32), tab_ref[...],
            dimension_numbers=(((1,), (0,)), ((), ())),
            preferred_element_type=jnp.float32)

    return pl.pallas_call(
        body,
        grid=(S_TC // BS_TC,),
        in_specs=[
            pl.BlockSpec((1, 1, BS_TC), lambda i: (i, 0, 0)),
            pl.BlockSpec((13, D_MODEL), lambda i: (0, 0)),
            pl.BlockSpec(memory_space=pl.ANY),
        ],
        out_specs=pl.BlockSpec((BS_TC, 1, D_MODEL), lambda i: (i, 0, 0)),
        out_shape=jax.ShapeDtypeStruct((BATCH, 1, D_MODEL), jnp.float32),
        input_output_aliases={2: 0},
    )(wk3, table, buf)


def kernel(weeks, table):
    wk = weeks.astype(jnp.int32)
    rep = _tc_replicate(table)
    buf = _sc_embed(rep, wk)
    wk3 = wk[:S_TC].reshape(S_TC // BS_TC, 1, BS_TC)
    return _tc_fill(wk3, table, buf)
